# baseline (speedup 1.0000x reference)
import jax
import jax.numpy as jnp
from jax import lax
from jax.experimental import pallas as pl
from jax.experimental.pallas import tpu as pltpu
from jax.experimental.pallas import tpu_sc as plsc

DIM = 64
NF = 50
B = 16384
NC = 2
NS = 16
NW = NC * NS
NE = 3 * B
EPW = NE // NW
EPAD = 1664
MAXB = EPW + 1
DUMP = NW * EPAD
STGROWS = NW * EPAD + 16
RING = 8
WKW = 256
CH = 128
NCH = (B // NW) // CH


def _lanesum(x):
    idx = jnp.arange(16, dtype=jnp.int32)
    dnums = lax.GatherDimensionNumbers(
        offset_dims=(), collapsed_slice_dims=(0,), start_index_map=(0,))
    for sh in (8, 4, 2, 1):
        perm = jnp.bitwise_xor(idx, sh)
        x = x + lax.gather(x, perm[:, None], dimension_numbers=dnums,
                           slice_sizes=(1,),
                           mode=lax.GatherScatterMode.PROMISE_IN_BOUNDS)
    return x


def _rsqrt16(x):
    i = plsc.bitcast(x, jnp.int32)
    i = jnp.int32(0x5F3759DF) - jnp.right_shift(i, 1)
    y = plsc.bitcast(i, jnp.float32)
    half = x * jnp.float32(0.5)
    for _ in range(3):
        y = y * (jnp.float32(1.5) - half * y * y)
    return y


def _b16(v):
    return jnp.full((16,), v, jnp.int32)


def _sweep_body(et_hbm, rt_hbm, blk_hbm, est_hbm, col_hbm, stg_hbm,
                blk_v, est_v, col_v, ring_v, outb_v, sem_blk, sem_out):
    wid = lax.axis_index("s") * NC + lax.axis_index("c")
    lanes = jnp.arange(16, dtype=jnp.int32)

    pltpu.sync_copy(blk_hbm.at[wid], blk_v)
    pltpu.sync_copy(est_hbm.at[wid], est_v)
    pltpu.sync_copy(col_hbm.at[wid], col_v)

    def getv2(ref, j):
        g = plsc.load_gather(ref, [_b16(lax.div(j, 128)),
                                   _b16(lax.rem(j, 128))])
        return g[0]

    nblk = getv2(blk_v, EPAD - 1)

    def drain_out():
        pltpu.make_async_copy(
            outb_v.at[pl.ds(0, 16), :], stg_hbm.at[_b16(DUMP)],
            sem_out).wait()

    def block_body(j, gcount):
        enc = getv2(blk_v, j)
        tab = lax.shift_right_logical(enc, 30)
        off = pl.multiple_of(jnp.bitwise_and(enc, (1 << 30) - 1), 128)
        slot = jnp.bitwise_and(j, RING - 1)
        dst = ring_v.at[pl.ds(pl.multiple_of(slot * 64, 64), 64), :]

        def cp_e(_):
            pltpu.async_copy(et_hbm.at[:, pl.ds(off, 128)], dst, sem_blk)
            return 0

        def cp_r(_):
            pltpu.async_copy(rt_hbm.at[:, pl.ds(off, 128)], dst, sem_blk)
            return 0

        lax.cond(tab == 0, cp_e, cp_r, 0)
        return gcount

    def drain_blk(n):
        def one(_, c):
            pltpu.make_async_copy(
                et_hbm.at[:, pl.ds(0, 128)],
                ring_v.at[pl.ds(0, 64), :], sem_blk).wait()
            return c
        lax.fori_loop(0, n, one, 0)

    def process_block(j, gcount):
        e0 = getv2(est_v, j)
        e1 = getv2(est_v, j + 1)
        ng = lax.shift_right_logical(e1 - e0 + 15, 4)
        rowbase = jnp.bitwise_and(j, RING - 1) * 64

        def grp(g, gc):
            base = e0 + 16 * g
            idv = _b16(base) + lanes
            cols = plsc.load_gather(
                col_v, [lax.shift_right_logical(idv, 7),
                        jnp.bitwise_and(idv, 127)])
            ssq = jnp.zeros((16,), jnp.float32)
            for d in range(DIM):
                c = plsc.load_gather(ring_v, [_b16(rowbase + d), cols])
                ssq = ssq + c * c
            inv = _rsqrt16(jnp.maximum(ssq, jnp.float32(1e-24)))

            def w8(_):
                drain_out()
                return 0

            lax.cond(gc >= 4, w8, lambda _: 0, 0)
            q = jnp.bitwise_and(gc, 3)
            qrow = _b16(q * 16) + lanes
            for d in range(DIM):
                c = plsc.load_gather(ring_v, [_b16(rowbase + d), cols])
                plsc.store_scatter(outb_v, [qrow, _b16(d)], c * inv)
            dest = _b16(wid * EPAD + base) + lanes
            dest = jnp.where(idv < _b16(e1), dest, _b16(DUMP))
            pltpu.async_copy(
                outb_v.at[pl.ds(pl.multiple_of(q * 16, 16), 16), :],
                stg_hbm.at[dest], sem_out)
            return gc + 1

        return lax.fori_loop(0, ng, grp, gcount)

    nsuper = lax.div(nblk + RING - 1, RING)

    def super_body(s, gcount):
        lo = s * RING
        hi = jnp.minimum(lo + RING, nblk)
        lax.fori_loop(lo, hi, block_body, 0)
        drain_blk(hi - lo)
        return lax.fori_loop(lo, hi, process_block, gcount)

    gtotal = lax.fori_loop(0, nsuper, super_body, 0)

    def final_drain(_, c):
        drain_out()
        return c
    lax.fori_loop(0, jnp.minimum(gtotal, 4), final_drain, 0)


def _score_body(stg_hbm, prow_hbm, wk_hbm, out_hbm,
                prow_v, hv, rv, tv, wkv, score_v, sem):
    wid = lax.axis_index("s") * NC + lax.axis_index("c")
    lanes = jnp.arange(16, dtype=jnp.int32)
    zero32 = jnp.zeros((32,), jnp.bfloat16)

    pltpu.sync_copy(prow_hbm.at[:, pl.ds(wid * NCH, NCH)], prow_v)
    pltpu.sync_copy(wk_hbm, wkv)

    def load_pack(ref, i):
        x = [ref[i, pl.ds(16 * k, 16)] for k in range(4)]
        return (plsc.pack(x[0], x[1], format=plsc.PackFormat.INTERLEAVED),
                plsc.pack(x[2], x[3], format=plsc.PackFormat.INTERLEAVED))

    for c in range(NCH):
        copies = [
            pltpu.async_copy(stg_hbm.at[prow_v.at[0, c]], hv, sem),
            pltpu.async_copy(stg_hbm.at[prow_v.at[1, c]], rv, sem),
            pltpu.async_copy(stg_hbm.at[prow_v.at[2, c]], tv, sem),
        ]
        for cp in copies:
            cp.wait()

        def pair(g, carry):
            i0 = 2 * g
            i1 = 2 * g + 1
            hp = (load_pack(hv, i0), load_pack(hv, i1))
            rp = (load_pack(rv, i0), load_pack(rv, i1))
            tp = (load_pack(tv, i0), load_pack(tv, i1))
            acc = [[jnp.zeros((16,), jnp.float32) for _ in range(4)]
                   for _ in range(2)]
            for f in range(NF):
                w = [wkv[f, pl.ds(32 * j, 32)] for j in range(2)]
                ka = wkv[f, pl.ds(64, 32)]
                kb = wkv[f, pl.ds(96, 32)]
                kc = wkv[f, pl.ds(128, 32)]
                for tt in range(2):
                    for j in range(2):
                        z = hp[tt][j] * ka + rp[tt][j] * kb + tp[tt][j] * kc
                        z = jnp.maximum(z, zero32)
                        p = z * w[j]
                        u0, u1 = plsc.unpack(
                            p, format=plsc.PackFormat.INTERLEAVED)
                        acc[tt][2 * j] = acc[tt][2 * j] + u0
                        acc[tt][2 * j + 1] = acc[tt][2 * j + 1] + u1
            tot0 = -_lanesum(acc[0][0] + acc[0][1] + acc[0][2] + acc[0][3])
            tot1 = -_lanesum(acc[1][0] + acc[1][1] + acc[1][2] + acc[1][3])
            val = jnp.where(lanes == 0, tot0, tot1)
            plsc.store_scatter(
                score_v, [_b16(c * CH + i0) + lanes], val, mask=lanes < 2)
            return carry

        lax.fori_loop(0, CH // 2, pair, 0)

    pltpu.sync_copy(score_v, out_hbm.at[pl.ds(wid * (B // NW), B // NW)])


def _interleave(a, b):
    return jnp.stack([a, b], axis=-1).reshape(a.shape[0], 32)


def kernel(T, E_table, R_table, kernel, fc_W):
    i_h = T[:, 0].astype(jnp.int32)
    i_r = T[:, 1].astype(jnp.int32)
    i_t = T[:, 2].astype(jnp.int32)
    idx_all = jnp.concatenate([i_h, i_t, i_r])
    tab = jnp.concatenate([jnp.zeros(2 * B, jnp.int32),
                           jnp.ones(B, jnp.int32)])
    blk = lax.shift_right_logical(idx_all, 7)
    col = jnp.bitwise_and(idx_all, 127)
    key = jnp.bitwise_or(lax.shift_left(tab, 13), blk)
    perm = jnp.argsort(key)
    skey = key[perm]
    scol = col[perm]

    pos = jnp.zeros((NE,), jnp.int32).at[perm].set(
        jnp.arange(NE, dtype=jnp.int32))
    prow = (pos // EPW) * EPAD + (pos % EPW)
    prow_h, prow_t, prow_r = prow[:B], prow[B:2 * B], prow[2 * B:]
    prow3 = jnp.stack([prow_h, prow_r, prow_t]).reshape(3, B // CH, CH)

    K2 = skey.reshape(NW, EPW)
    first = jnp.concatenate(
        [jnp.ones((NW, 1), bool), K2[:, 1:] != K2[:, :-1]], axis=1)
    nblk_w = first.sum(axis=1).astype(jnp.int32)
    bpos = jnp.cumsum(first, axis=1).astype(jnp.int32) - 1
    stab = lax.shift_right_logical(K2, 13)
    soff = lax.shift_left(jnp.bitwise_and(K2, (1 << 13) - 1), 7)
    enc = jnp.bitwise_or(soff, lax.shift_left(stab, 30))

    wrow = jnp.repeat(jnp.arange(NW, dtype=jnp.int32)[:, None], EPW, axis=1)
    blk_arr = jnp.zeros((NW, EPAD), jnp.int32)
    blk_arr = blk_arr.at[wrow.ravel(), bpos.ravel()].set(enc.ravel())
    blk_arr = blk_arr.at[:, EPAD - 1].set(nblk_w)
    est_arr = jnp.full((NW, EPAD), EPW, jnp.int32)
    eloc = jnp.repeat(jnp.arange(EPW, dtype=jnp.int32)[None, :], NW, axis=0)
    est_arr = est_arr.at[wrow.ravel(), jnp.where(
        first, bpos, MAXB).ravel()].set(
        jnp.where(first, eloc, EPW).ravel(), mode="drop")
    col_arr = jnp.pad(scol.reshape(NW, EPW), ((0, 0), (0, EPAD - EPW)))

    blk3 = blk_arr.reshape(NW, EPAD // 128, 128)
    est3 = est_arr.reshape(NW, EPAD // 128, 128)
    col3 = col_arr.reshape(NW, EPAD // 128, 128)

    k3 = kernel[:, 0, 0, :].astype(jnp.bfloat16)
    kb3 = jnp.repeat(k3[:, :, None], 32, axis=2)
    W = fc_W.reshape(NF, DIM).astype(jnp.bfloat16)
    w0 = _interleave(W[:, 0:16], W[:, 16:32])
    w1 = _interleave(W[:, 32:48], W[:, 48:64])
    wk = jnp.concatenate(
        [w0, w1, kb3[:, 0], kb3[:, 1], kb3[:, 2],
         jnp.zeros((NF, WKW - 160), jnp.bfloat16)], axis=1)

    et = E_table.T
    rt = R_table.T

    mesh = plsc.VectorSubcoreMesh(core_axis_name="c", subcore_axis_name="s")
    sweep = pl.kernel(
        _sweep_body,
        out_type=jax.ShapeDtypeStruct((STGROWS, 128), jnp.float32),
        mesh=mesh,
        compiler_params=pltpu.CompilerParams(needs_layout_passes=False,
                                             use_tc_tiling_on_sc=True),
        scratch_types=[
            pltpu.VMEM((EPAD // 128, 128), jnp.int32),
            pltpu.VMEM((EPAD // 128, 128), jnp.int32),
            pltpu.VMEM((EPAD // 128, 128), jnp.int32),
            pltpu.VMEM((RING * 64, 128), jnp.float32),
            pltpu.VMEM((64, 128), jnp.float32),
            pltpu.SemaphoreType.DMA,
            pltpu.SemaphoreType.DMA,
        ],
    )
    stg = sweep(et, rt, blk3, est3, col3)

    score = pl.kernel(
        _score_body,
        out_type=jax.ShapeDtypeStruct((B,), jnp.float32),
        mesh=mesh,
        compiler_params=pltpu.CompilerParams(needs_layout_passes=False,
                                             use_tc_tiling_on_sc=False),
        scratch_types=[
            pltpu.VMEM((3, NCH, CH), jnp.int32),
            pltpu.VMEM((CH, 128), jnp.float32),
            pltpu.VMEM((CH, 128), jnp.float32),
            pltpu.VMEM((CH, 128), jnp.float32),
            pltpu.VMEM((NF, WKW), jnp.bfloat16),
            pltpu.VMEM((B // NW,), jnp.float32),
            pltpu.SemaphoreType.DMA,
        ],
    )
    return score(stg, prow3, wk)


# speedup vs baseline: 6.7971x; 6.7971x over previous
import jax
import jax.numpy as jnp
from jax import lax
from jax.experimental import pallas as pl
from jax.experimental.pallas import tpu as pltpu
from jax.experimental.pallas import tpu_sc as plsc

DIM = 64
NF = 50
B = 16384
NC = 2
NS = 16
NW = NC * NS
BPW = B // NW
CHUNK = 128
NCHUNK = BPW // CHUNK


def _lanesum(x):
    idx = jnp.arange(16, dtype=jnp.int32)
    dnums = lax.GatherDimensionNumbers(
        offset_dims=(), collapsed_slice_dims=(0,), start_index_map=(0,))
    for sh in (8, 4, 2, 1):
        perm = jnp.bitwise_xor(idx, sh)
        x = x + lax.gather(x, perm[:, None], dimension_numbers=dnums,
                           slice_sizes=(1,),
                           mode=lax.GatherScatterMode.PROMISE_IN_BOUNDS)
    return x


def _rsqrt16(x):
    i = plsc.bitcast(x, jnp.int32)
    i = jnp.int32(0x5F3759DF) - jnp.right_shift(i, 1)
    y = plsc.bitcast(i, jnp.float32)
    half = x * jnp.float32(0.5)
    for _ in range(3):
        y = y * (jnp.float32(1.5) - half * y * y)
    return y


def _body(idx_hbm, e_hbm, r_hbm, wk_hbm, out_hbm,
          idx_v, hv, rv, tv, wkv, score_v, sem):
    wid = lax.axis_index("s") * NC + lax.axis_index("c")
    base_row = wid * NCHUNK

    pltpu.sync_copy(idx_hbm.at[:, pl.ds(base_row, NCHUNK)], idx_v)
    pltpu.sync_copy(wk_hbm, wkv)

    def issue(c, slot):
        ds = pl.ds(slot * CHUNK, CHUNK)
        return [
            pltpu.async_copy(e_hbm.at[idx_v.at[0, c]], hv.at[ds], sem),
            pltpu.async_copy(r_hbm.at[idx_v.at[1, c]], rv.at[ds], sem),
            pltpu.async_copy(e_hbm.at[idx_v.at[2, c]], tv.at[ds], sem),
        ]

    lane0 = jnp.arange(16, dtype=jnp.int32) == 0

    def chunk_compute(c, slot):
        def triple(i, carry):
            row = slot * CHUNK + i
            h = [hv[row, pl.ds(16 * k, 16)] for k in range(4)]
            r = [rv[row, pl.ds(16 * k, 16)] for k in range(4)]
            t = [tv[row, pl.ds(16 * k, 16)] for k in range(4)]

            def inv_norm(x):
                ssq = x[0] * x[0] + x[1] * x[1] + x[2] * x[2] + x[3] * x[3]
                s = _lanesum(ssq)
                return _rsqrt16(jnp.maximum(s, jnp.float32(1e-24)))

            ih, ir, it = inv_norm(h), inv_norm(r), inv_norm(t)
            h = [x * ih for x in h]
            r = [x * ir for x in r]
            t = [x * it for x in t]

            acc = [jnp.zeros((16,), jnp.float32) for _ in range(4)]
            for f in range(NF):
                ka = wkv[f, pl.ds(DIM, 16)]
                kb = wkv[f, pl.ds(DIM + 16, 16)]
                kc = wkv[f, pl.ds(DIM + 32, 16)]
                for k in range(4):
                    z = h[k] * ka + r[k] * kb + t[k] * kc
                    z = jnp.maximum(z, jnp.float32(0.0))
                    acc[k] = acc[k] + z * wkv[f, pl.ds(16 * k, 16)]
            tot = -_lanesum(acc[0] + acc[1] + acc[2] + acc[3])
            plsc.store_scatter(score_v,
                               [jnp.full((16,), c * CHUNK + i, jnp.int32)],
                               tot, mask=lane0)
            return carry

        lax.fori_loop(0, CHUNK, triple, 0)

    cur = issue(0, 0)
    for c in range(NCHUNK):
        for cp in cur:
            cp.wait()
        if c + 1 < NCHUNK:
            cur = issue(c + 1, (c + 1) & 1)
        chunk_compute(c, c & 1)

    pltpu.sync_copy(score_v, out_hbm.at[pl.ds(wid * BPW, BPW)])


def kernel(T, E_table, R_table, kernel, fc_W):
    idx = T.T.reshape(3, B // CHUNK, CHUNK).astype(jnp.int32)
    k3 = kernel[:, 0, 0, :]
    kbt = jnp.repeat(k3, 16, axis=1).astype(jnp.float32)
    W = fc_W.reshape(NF, DIM)
    wk = jnp.concatenate(
        [W, kbt, jnp.zeros((NF, 128 - DIM - 48), jnp.float32)], axis=1)

    e128 = jnp.pad(E_table, ((0, 0), (0, 128 - DIM)))
    r128 = jnp.pad(R_table, ((0, 0), (0, 128 - DIM)))

    mesh = plsc.VectorSubcoreMesh(core_axis_name="c", subcore_axis_name="s")
    run = pl.kernel(
        _body,
        out_type=jax.ShapeDtypeStruct((B,), jnp.float32),
        mesh=mesh,
        compiler_params=pltpu.CompilerParams(needs_layout_passes=False,
                                             use_tc_tiling_on_sc=False),
        scratch_types=[
            pltpu.VMEM((3, NCHUNK, CHUNK), jnp.int32),
            pltpu.VMEM((2 * CHUNK, 128), jnp.float32),
            pltpu.VMEM((2 * CHUNK, 128), jnp.float32),
            pltpu.VMEM((2 * CHUNK, 128), jnp.float32),
            pltpu.VMEM((NF, 128), jnp.float32),
            pltpu.VMEM((BPW,), jnp.float32),
            pltpu.SemaphoreType.DMA,
        ],
    )
    return run(idx, e128, r128, wk)


# speedup vs baseline: 9.2033x; 1.3540x over previous
import jax
import jax.numpy as jnp
from jax import lax
from jax.experimental import pallas as pl
from jax.experimental.pallas import tpu as pltpu
from jax.experimental.pallas import tpu_sc as plsc

DIM = 64
NF = 50
B = 16384
NC = 2
NS = 16
NW = NC * NS
BPW = B // NW
CHUNK = 128
NCHUNK = BPW // CHUNK
RING = 8


def _lanesum(x):
    idx = jnp.arange(16, dtype=jnp.int32)
    dnums = lax.GatherDimensionNumbers(
        offset_dims=(), collapsed_slice_dims=(0,), start_index_map=(0,))
    for sh in (8, 4, 2, 1):
        perm = jnp.bitwise_xor(idx, sh)
        x = x + lax.gather(x, perm[:, None], dimension_numbers=dnums,
                           slice_sizes=(1,),
                           mode=lax.GatherScatterMode.PROMISE_IN_BOUNDS)
    return x


def _rsqrt16(x):
    i = plsc.bitcast(x, jnp.int32)
    i = jnp.int32(0x5F3759DF) - jnp.right_shift(i, 1)
    y = plsc.bitcast(i, jnp.float32)
    half = x * jnp.float32(0.5)
    for _ in range(3):
        y = y * (jnp.float32(1.5) - half * y * y)
    return y


def _b16(v):
    return jnp.full((16,), v, jnp.int32)


def _body(idx_hbm, e_hbm, r_hbm, wk_hbm, out_hbm,
          idx_v, hr, rr, tr, wkv, score_v, sem):
    wid = lax.axis_index("s") * NC + lax.axis_index("c")
    base_row = wid * NCHUNK
    lanes = jnp.arange(16, dtype=jnp.int32)

    pltpu.sync_copy(idx_hbm.at[:, pl.ds(base_row, NCHUNK)], idx_v)
    pltpu.sync_copy(wk_hbm, wkv)

    s012 = jnp.minimum(lanes, 2)

    def idx3(i):
        g = plsc.load_gather(
            idx_v, [s012, _b16(lax.div(i, CHUNK)), _b16(lax.rem(i, CHUNK))])
        return g[0], g[1], g[2]

    def issue(j):
        ih, ir_, it = idx3(j)
        slot = jnp.bitwise_and(j, RING - 1) * 8
        ds = pl.ds(pl.multiple_of(slot, 8), 8)
        pltpu.async_copy(
            e_hbm.at[pl.ds(pl.multiple_of(
                lax.shift_left(lax.shift_right_logical(ih, 3), 3), 8), 8)],
            hr.at[ds], sem)
        pltpu.async_copy(
            r_hbm.at[pl.ds(pl.multiple_of(
                lax.shift_left(lax.shift_right_logical(ir_, 3), 3), 8), 8)],
            rr.at[ds], sem)
        pltpu.async_copy(
            e_hbm.at[pl.ds(pl.multiple_of(
                lax.shift_left(lax.shift_right_logical(it, 3), 3), 8), 8)],
            tr.at[ds], sem)

    def drain3():
        for ref in (hr, rr, tr):
            pltpu.make_async_copy(
                e_hbm.at[pl.ds(0, 8)], ref.at[pl.ds(0, 8)], sem).wait()

    for j in range(RING):
        issue(j)

    lane0 = lanes == 0

    def triple(i, carry):
        drain3()
        ih, ir_, it = idx3(i)
        slot = jnp.bitwise_and(i, RING - 1) * 8

        def rows(sub, ref):
            base = slot + sub
            return [plsc.load_gather(ref, [_b16(base), _b16(16 * k) + lanes])
                    for k in range(4)]

        h = rows(jnp.bitwise_and(ih, 7), hr)
        r = rows(jnp.bitwise_and(ir_, 7), rr)
        t = rows(jnp.bitwise_and(it, 7), tr)

        def nxt(_):
            issue(i + RING)
            return 0

        lax.cond(i + RING < BPW, nxt, lambda _: 0, 0)

        def inv_norm(x):
            ssq = x[0] * x[0] + x[1] * x[1] + x[2] * x[2] + x[3] * x[3]
            s = _lanesum(ssq)
            return _rsqrt16(jnp.maximum(s, jnp.float32(1e-24)))

        ihn, irn, itn = inv_norm(h), inv_norm(r), inv_norm(t)
        h = [x * ihn for x in h]
        r = [x * irn for x in r]
        t = [x * itn for x in t]

        acc = [jnp.zeros((16,), jnp.float32) for _ in range(4)]
        for f in range(NF):
            ka = wkv[f, pl.ds(DIM, 16)]
            kb = wkv[f, pl.ds(DIM + 16, 16)]
            kc = wkv[f, pl.ds(DIM + 32, 16)]
            for k in range(4):
                z = h[k] * ka + r[k] * kb + t[k] * kc
                z = jnp.maximum(z, jnp.float32(0.0))
                acc[k] = acc[k] + z * wkv[f, pl.ds(16 * k, 16)]
        tot = -_lanesum(acc[0] + acc[1] + acc[2] + acc[3])
        plsc.store_scatter(score_v, [_b16(i)], tot, mask=lane0)
        return carry

    lax.fori_loop(0, BPW, triple, 0)
    pltpu.sync_copy(score_v, out_hbm.at[pl.ds(wid * BPW, BPW)])


def kernel(T, E_table, R_table, kernel, fc_W):
    idx = T.T.reshape(3, B // CHUNK, CHUNK).astype(jnp.int32)
    k3 = kernel[:, 0, 0, :]
    kbt = jnp.repeat(k3, 16, axis=1).astype(jnp.float32)
    W = fc_W.reshape(NF, DIM)
    wk = jnp.concatenate(
        [W, kbt, jnp.zeros((NF, 128 - DIM - 48), jnp.float32)], axis=1)

    mesh = plsc.VectorSubcoreMesh(core_axis_name="c", subcore_axis_name="s")
    run = pl.kernel(
        _body,
        out_type=jax.ShapeDtypeStruct((B,), jnp.float32),
        mesh=mesh,
        compiler_params=pltpu.CompilerParams(needs_layout_passes=False,
                                             use_tc_tiling_on_sc=True),
        scratch_types=[
            pltpu.VMEM((3, NCHUNK, CHUNK), jnp.int32),
            pltpu.VMEM((RING * 8, DIM), jnp.float32),
            pltpu.VMEM((RING * 8, DIM), jnp.float32),
            pltpu.VMEM((RING * 8, DIM), jnp.float32),
            pltpu.VMEM((NF, 128), jnp.float32),
            pltpu.VMEM((BPW,), jnp.float32),
            pltpu.SemaphoreType.DMA,
        ],
    )
    return run(idx, E_table, R_table, wk)
